# baseline (speedup 1.0000x reference)
import functools

import jax
import jax.numpy as jnp
from jax import lax
from jax.experimental import pallas as pl
from jax.experimental.pallas import tpu as pltpu
from jax.experimental.pallas import tpu_sc as plsc


def _sc_gather(table_pad, idx_flat, n_idx, width):
    info = plsc.get_sparse_core_info()
    nw = info.num_cores * info.num_subcores
    n_per_w = n_idx // nw
    mesh = plsc.VectorSubcoreMesh(core_axis_name="c", subcore_axis_name="s")

    @functools.partial(
        pl.kernel,
        mesh=mesh,
        out_type=jax.ShapeDtypeStruct((n_idx, width), jnp.float32),
        scratch_types=[
            pltpu.VMEM((n_per_w,), jnp.int32),
            pltpu.VMEM((n_per_w, width), jnp.float32),
            pltpu.SemaphoreType.DMA,
        ],
    )
    def gather_kernel(table_hbm, idx_hbm, out_hbm, idx_v, rows_v, sem):
        wid = lax.axis_index("s") * info.num_cores + lax.axis_index("c")
        base = wid * n_per_w
        pltpu.sync_copy(idx_hbm.at[pl.ds(base, n_per_w)], idx_v)
        pltpu.async_copy(table_hbm.at[idx_v], rows_v, sem).wait()
        pltpu.sync_copy(rows_v, out_hbm.at[pl.ds(base, n_per_w)])

    return gather_kernel(table_pad, idx_flat)


def _add_body(x_ref, pr_ref, pc_ref, o_ref):
    x = x_ref[0]
    rb, l, dim = x.shape
    pr = pr_ref[0]
    pc = pc_ref[0]
    o_ref[0] = x + pr[:, None, :] + pc[None, :, :]


def _tc_add(x, pe2, row_block):
    b, l, _, dim = x.shape
    grid = (b, l // row_block)
    return pl.pallas_call(
        _add_body,
        grid=grid,
        in_specs=[
            pl.BlockSpec((1, row_block, l, dim), lambda i, r: (i, r, 0, 0)),
            pl.BlockSpec((1, row_block, dim), lambda i, r: (i, r, 0)),
            pl.BlockSpec((1, l, dim), lambda i, r: (i, 0, 1)),
        ],
        out_specs=pl.BlockSpec((1, row_block, l, dim), lambda i, r: (i, r, 0, 0)),
        out_shape=jax.ShapeDtypeStruct(x.shape, x.dtype),
    )(x, pe2, pe2)


def kernel(x, aa_idx, pos_enc):
    b, l, _, dim = x.shape
    dh = dim // 2
    v = pos_enc.shape[0]
    idx_flat = aa_idx.reshape(-1).astype(jnp.int32)
    zeros = jnp.zeros((v, dim), pos_enc.dtype)
    table2 = jnp.concatenate([pos_enc, zeros, pos_enc], axis=1)
    pe2 = _sc_gather(table2, idx_flat, b * l, 2 * dim)
    pe2 = pe2.reshape(b, l, 2 * dim)
    return _tc_add(x, pe2, 16)


# speedup vs baseline: 1.0151x; 1.0151x over previous
import functools

import jax
import jax.numpy as jnp
from jax import lax
from jax.experimental import pallas as pl
from jax.experimental.pallas import tpu as pltpu
from jax.experimental.pallas import tpu_sc as plsc


def _sc_gather(table_pad, idx_flat, n_idx, width):
    info = plsc.get_sparse_core_info()
    nw = info.num_cores * info.num_subcores
    n_per_w = n_idx // nw
    mesh = plsc.VectorSubcoreMesh(core_axis_name="c", subcore_axis_name="s")

    @functools.partial(
        pl.kernel,
        mesh=mesh,
        out_type=jax.ShapeDtypeStruct((n_idx, width), jnp.float32),
        scratch_types=[
            pltpu.VMEM((n_per_w,), jnp.int32),
            pltpu.VMEM((n_per_w, width), jnp.float32),
            pltpu.SemaphoreType.DMA,
        ],
    )
    def gather_kernel(table_hbm, idx_hbm, out_hbm, idx_v, rows_v, sem):
        wid = lax.axis_index("s") * info.num_cores + lax.axis_index("c")
        base = wid * n_per_w
        pltpu.sync_copy(idx_hbm.at[pl.ds(base, n_per_w)], idx_v)
        pltpu.async_copy(table_hbm.at[idx_v], rows_v, sem).wait()
        pltpu.sync_copy(rows_v, out_hbm.at[pl.ds(base, n_per_w)])

    return gather_kernel(table_pad, idx_flat)


def _add_body(x_ref, pr_ref, pc_ref, o_ref):
    x = x_ref[0]
    rb, l, dim = x.shape
    dh = dim // 2
    pr = pr_ref[0]
    pc = pc_ref[0]
    col = jnp.concatenate([pc[:, dh:], pc[:, :dh]], axis=-1)
    o_ref[0] = x + pr[:, None, :] + col[None, :, :]


def _tc_add(x, pe_pad, row_block):
    b, l, _, dim = x.shape
    grid = (b, l // row_block)
    return pl.pallas_call(
        _add_body,
        grid=grid,
        in_specs=[
            pl.BlockSpec((1, row_block, l, dim), lambda i, r: (i, r, 0, 0)),
            pl.BlockSpec((1, row_block, dim), lambda i, r: (i, r, 0)),
            pl.BlockSpec((1, l, dim), lambda i, r: (i, 0, 0)),
        ],
        out_specs=pl.BlockSpec((1, row_block, l, dim), lambda i, r: (i, r, 0, 0)),
        out_shape=jax.ShapeDtypeStruct(x.shape, x.dtype),
    )(x, pe_pad, pe_pad)


def kernel(x, aa_idx, pos_enc):
    b, l, _, dim = x.shape
    dh = dim // 2
    idx_flat = aa_idx.reshape(-1).astype(jnp.int32)
    table_pad = jnp.pad(pos_enc, ((0, 0), (0, dim - dh)))
    pe_pad = _sc_gather(table_pad, idx_flat, b * l, dim)
    pe_pad = pe_pad.reshape(b, l, dim)
    return _tc_add(x, pe_pad, 32)


# speedup vs baseline: 1.0173x; 1.0021x over previous
import functools

import jax
import jax.numpy as jnp
from jax import lax
from jax.experimental import pallas as pl
from jax.experimental.pallas import tpu as pltpu
from jax.experimental.pallas import tpu_sc as plsc


def _sc_gather(table_pad, idx_flat, n_idx, width):
    info = plsc.get_sparse_core_info()
    nw = info.num_cores * info.num_subcores
    n_per_w = n_idx // nw
    mesh = plsc.VectorSubcoreMesh(core_axis_name="c", subcore_axis_name="s")

    @functools.partial(
        pl.kernel,
        mesh=mesh,
        out_type=jax.ShapeDtypeStruct((n_idx, width), jnp.float32),
        scratch_types=[
            pltpu.VMEM((n_per_w,), jnp.int32),
            pltpu.VMEM((n_per_w, width), jnp.float32),
            pltpu.SemaphoreType.DMA,
        ],
    )
    def gather_kernel(table_hbm, idx_hbm, out_hbm, idx_v, rows_v, sem):
        wid = lax.axis_index("s") * info.num_cores + lax.axis_index("c")
        base = wid * n_per_w
        pltpu.sync_copy(idx_hbm.at[pl.ds(base, n_per_w)], idx_v)
        pltpu.async_copy(table_hbm.at[idx_v], rows_v, sem).wait()
        pltpu.sync_copy(rows_v, out_hbm.at[pl.ds(base, n_per_w)])

    return gather_kernel(table_pad, idx_flat)


def _add_body(x_ref, pc_ref, o_ref):
    x = x_ref[0]
    rb, l, dim = x.shape
    dh = dim // 2
    r = pl.program_id(1)
    pc = pc_ref[0]
    pr = pc_ref[0, pl.ds(r * rb, rb), :]
    col = jnp.concatenate([pc[:, dh:], pc[:, :dh]], axis=-1)
    o_ref[0] = x + pr[:, None, :] + col[None, :, :]


def _tc_add(x, pe_pad, row_block):
    b, l, _, dim = x.shape
    grid = (b, l // row_block)
    return pl.pallas_call(
        _add_body,
        grid=grid,
        in_specs=[
            pl.BlockSpec((1, row_block, l, dim), lambda i, r: (i, r, 0, 0)),
            pl.BlockSpec((1, l, dim), lambda i, r: (i, 0, 0)),
        ],
        out_specs=pl.BlockSpec((1, row_block, l, dim), lambda i, r: (i, r, 0, 0)),
        out_shape=jax.ShapeDtypeStruct(x.shape, x.dtype),
    )(x, pe_pad)


def kernel(x, aa_idx, pos_enc):
    b, l, _, dim = x.shape
    dh = dim // 2
    idx_flat = aa_idx.reshape(-1).astype(jnp.int32)
    table_pad = jnp.pad(pos_enc, ((0, 0), (0, dim - dh)))
    pe_pad = _sc_gather(table_pad, idx_flat, b * l, dim)
    pe_pad = pe_pad.reshape(b, l, dim)
    return _tc_add(x, pe_pad, 32)
